# baseline (speedup 1.0000x reference)
import jax
import jax.numpy as jnp
from jax import lax
from jax.experimental import pallas as pl
from jax.experimental.pallas import tpu as pltpu
from jax.experimental.pallas import tpu_sc as plsc

N = 65536
D = 128
B = 8192
NC = 2
NS = 16
L = 16
NW = NC * NS
ROWS_PER_W = N // NW
POS_PER_TILE = B // NS
CH = 128
NCH = POS_PER_TILE // CH
GPOS = B // NW
GCH = GPOS // CH
NK = D // L
LSZ = ROWS_PER_W + 2 * CH
PW = 512


def _body_a(x_hbm, idx0_hbm, idx1_hbm,
            part_hbm, t0_hbm, t1_hbm,
            idx0_v, idx1_v, g0i_v, g1i_v, ones_v, rows2_v, f0_v,
            sel_v, partial_v, pall_v, accv_v, zbuf_v, zbuf2_v,
            table0, table1, accspm, pspm, semg0, semg1, sem2):
    c = lax.axis_index("c")
    s = lax.axis_index("s")
    wid = s * NC + c

    def zb(i, _):
        zbuf_v[pl.ds(i * L, L)] = jnp.zeros((L,), jnp.float32)
        return 0
    lax.fori_loop(0, (N // NS) // L, zb, 0, unroll=4)
    pltpu.sync_copy(zbuf_v, table0.at[pl.ds(s * (N // NS), N // NS)])
    pltpu.sync_copy(zbuf_v, table1.at[pl.ds(s * (N // NS), N // NS)])
    def zb2(i, _):
        for k in range(NK):
            zbuf2_v[i, pl.ds(k * L, L)] = jnp.zeros((L,), jnp.float32)
        return 0
    lax.fori_loop(0, L, zb2, 0)
    for r in range(3):
        pltpu.sync_copy(zbuf2_v, accspm.at[pl.ds(48 * s + 16 * r, 16)])
    for k in range(CH // L):
        ones_v[pl.ds(k * L, L)] = jnp.ones((L,), jnp.float32)
    plsc.subcore_barrier()

    gbase = B // NC * c + GPOS * s
    idx_copies = []
    for j in range(NCH):
        idx_copies.append((idx0_hbm.at[pl.ds(POS_PER_TILE * s + CH * j, CH)],
                           idx0_v.at[j]))
        idx_copies.append((idx1_hbm.at[pl.ds(POS_PER_TILE * s + CH * j, CH)],
                           idx1_v.at[j]))
    for j in range(GCH):
        idx_copies.append((idx0_hbm.at[pl.ds(gbase + CH * j, CH)],
                           g0i_v.at[j]))
        idx_copies.append((idx1_hbm.at[pl.ds(gbase + CH * j, CH)],
                           g1i_v.at[j]))
    for src, dst in idx_copies:
        pltpu.async_copy(src, dst, semg0)
    for src, dst in idx_copies:
        pltpu.make_async_copy(src, dst, semg0).wait()

    add_copies = []
    for j in range(NCH):
        add_copies.append((ones_v, table0.at[idx0_v.at[j]]))
        add_copies.append((ones_v, table1.at[idx1_v.at[j]]))
    for src, dst in add_copies:
        pltpu.async_copy(src, dst, semg1, add=True)
    for src, dst in add_copies:
        pltpu.make_async_copy(src, dst, semg1).wait()
    plsc.subcore_barrier()

    pltpu.async_copy(table0.at[pl.ds(wid * ROWS_PER_W, ROWS_PER_W)],
                     t0_hbm.at[pl.ds(wid * ROWS_PER_W, ROWS_PER_W)], sem2)
    pltpu.async_copy(table1.at[pl.ds(wid * ROWS_PER_W, ROWS_PER_W)],
                     t1_hbm.at[pl.ds(wid * ROWS_PER_W, ROWS_PER_W)], sem2)
    sems = [semg0, semg1]

    lanei = lax.iota(jnp.int32, L)
    for k in range(CH // L):
        sel_v[0, pl.ds(k * L, L)] = jnp.full((L,), 48 * s, jnp.int32) + lanei
    for j in range(GCH):
        pltpu.async_copy(x_hbm.at[g0i_v.at[j]], rows2_v.at[j], sems[j])
    for j in range(GCH):
        pltpu.make_async_copy(x_hbm.at[g0i_v.at[j]],
                              rows2_v.at[j], sems[j]).wait()
        pltpu.sync_copy(rows2_v.at[j], accspm.at[sel_v.at[0]], add=True)

    for j in range(GCH):
        pltpu.async_copy(x_hbm.at[g1i_v.at[j]], rows2_v.at[j], sems[j])
    cvec = jnp.zeros((L,), jnp.float32)
    one = jnp.ones((L,), jnp.float32)
    zerov = jnp.zeros((L,), jnp.float32)
    hbase = jnp.full((L,), 48 * s + 16, jnp.int32) + lanei
    mbase = jnp.full((L,), 48 * s + 32, jnp.int32) + lanei
    for j in range(GCH):
        pltpu.sync_copy(table0.at[g1i_v.at[j]], f0_v.at[pl.ds(0, CH)])
        for k in range(CH // L):
            hit = f0_v[pl.ds(k * L, L)] > 0.0
            sel_v[1, pl.ds(k * L, L)] = jnp.where(hit, hbase, mbase)
            cvec = cvec + jnp.where(hit, one, zerov)
        pltpu.make_async_copy(x_hbm.at[g1i_v.at[j]],
                              rows2_v.at[j], sems[j]).wait()
        pltpu.sync_copy(rows2_v.at[j], accspm.at[sel_v.at[1]], add=True)

    pltpu.sync_copy(accspm.at[pl.ds(48 * s, 48)], accv_v)

    def redt(t, carry):
        res = []
        for g in range(3):
            for k in range(NK):
                res.append(carry[g * NK + k]
                           + accv_v[16 * g + t, pl.ds(k * L, L)])
        return tuple(res)
    tot = lax.fori_loop(0, L, redt,
                        tuple([jnp.zeros((L,), jnp.float32)] * (3 * NK)),
                        unroll=2)
    for k in range(NK):
        partial_v[pl.ds(k * L, L)] = tot[k]
        partial_v[pl.ds(D + k * L, L)] = tot[NK + k] + tot[2 * NK + k]
        partial_v[pl.ds(2 * D + k * L, L)] = tot[NK + k]
    partial_v[pl.ds(3 * D, L)] = cvec
    partial_v[pl.ds(3 * D + L, L)] = jnp.zeros((L,), jnp.float32)
    pltpu.sync_copy(partial_v, pspm.at[s])
    plsc.subcore_barrier()

    @pl.when(s == 0)
    def _():
        pltpu.sync_copy(pspm, pall_v)

        def redp(t, carry):
            return tuple(carry[k] + pall_v[t, pl.ds(k * L, L)]
                         for k in range(3 * NK + 1))
        tot2 = lax.fori_loop(0, NS, redp,
                             tuple([jnp.zeros((L,), jnp.float32)]
                                   * (3 * NK + 1)), unroll=2)
        for k in range(3 * NK + 1):
            partial_v[pl.ds(k * L, L)] = tot2[k]
        partial_v[pl.ds((3 * NK + 1) * L, L)] = jnp.zeros((L,), jnp.float32)
        pltpu.sync_copy(partial_v, part_hbm.at[c])

    pltpu.make_async_copy(
        table0.at[pl.ds(wid * ROWS_PER_W, ROWS_PER_W)],
        t0_hbm.at[pl.ds(wid * ROWS_PER_W, ROWS_PER_W)], sem2).wait()
    pltpu.make_async_copy(
        table1.at[pl.ds(wid * ROWS_PER_W, ROWS_PER_W)],
        t1_hbm.at[pl.ds(wid * ROWS_PER_W, ROWS_PER_W)], sem2).wait()


def _body_b(x_hbm, part_hbm, t0_hbm, t1_hbm, out_hbm,
            rows2_v, pall2_v, fl0_v, fl1_v, lst_v, idxw_v,
            semg0, semg1, semg2, semg3, sems0, sems1, sems2, sems3, semprt):
    c = lax.axis_index("c")
    s = lax.axis_index("s")
    wid = s * NC + c
    base = wid * ROWS_PER_W

    fl_copies = [
        (t0_hbm.at[pl.ds(base, ROWS_PER_W)], fl0_v.at[pl.ds(0, ROWS_PER_W)]),
        (t1_hbm.at[pl.ds(base, ROWS_PER_W)], fl1_v.at[pl.ds(0, ROWS_PER_W)]),
    ]
    pltpu.async_copy(part_hbm, pall2_v, semprt)
    for src, dst in fl_copies:
        pltpu.async_copy(src, dst, semg0)
    for src, dst in fl_copies:
        pltpu.make_async_copy(src, dst, semg0).wait()

    lane = lax.iota(jnp.int32, L)

    def bld(g, cnt):
        f0 = fl0_v[pl.ds(g * L, L)]
        f1 = fl1_v[pl.ds(g * L, L)]
        m = (f0 > 0.0) | (f1 > 0.0)
        key = jnp.where(m, 0, 1).astype(jnp.int32)
        ids = jnp.full((L,), base + g * L, jnp.int32) + lane
        val = ((ids << 2)
               | jnp.where(f0 > 0.0, 2, 0).astype(jnp.int32)
               | jnp.where(f1 > 0.0, 1, 0).astype(jnp.int32))
        _, vs = plsc.sort_key_val(key, val)
        lst_v[pl.ds(cnt, L)] = vs
        return cnt + plsc.all_reduce_population_count(m)[0]
    with jax.named_scope("bld"):
        cnt = lax.fori_loop(0, ROWS_PER_W // L, bld, jnp.int32(0), unroll=2)

    lastp = jnp.maximum(cnt - 1, 0)
    lid = jnp.full((L,), lst_v[pl.ds(lastp, L)][0], jnp.int32)
    for k in range(NK):
        lst_v[pl.ds(cnt + k * L, L)] = lid

    trip = lax.shift_right_logical(cnt + (CH - 1), 7)
    semg = [semg0, semg1, semg2, semg3]
    semsc = [sems0, sems1, sems2, sems3]

    def issue(ch, p):
        off = ch * CH
        for k in range(NK):
            idxw_v[p, pl.ds(k * L, L)] = lax.shift_right_logical(
                lst_v[pl.ds(off + k * L, L)], 2)
        pltpu.async_copy(x_hbm.at[idxw_v.at[p]], rows2_v.at[p], semg[p])

    @pl.when(trip > 0)
    def _():
        issue(0, 0)

    pltpu.make_async_copy(part_hbm, pall2_v, semprt).wait()

    def two(off):
        return pall2_v[0, pl.ds(off, L)] + pall2_v[1, pl.ds(off, L)]
    sum0 = [two(k * L) for k in range(NK)]
    s1x = [two(D + k * L) for k in range(NK)]
    sh = [two(2 * D + k * L) for k in range(NK)]
    cl = two(3 * D)
    cv = jnp.full((L,), lax.reduce_sum(cl, axes=(0,)), jnp.float32)

    inv_b = jnp.float32(1.0 / B)
    m0 = [sum0[k] * inv_b for k in range(NK)]
    ilo8 = lax.iota(jnp.int32, L) < 8
    zero = jnp.zeros((L,), jnp.float32)
    m1 = []
    for k in range(NK):
        if k == 0:
            r = jnp.where(ilo8, sh[0], cv * m0[0])
        elif k == 1:
            r = jnp.where(ilo8, zero, cv * m0[1])
        else:
            r = cv * m0[k]
        m1.append((s1x[k] - sh[k] + r) * inv_b)
    rowz1 = jnp.where(ilo8, zero, m0[1])
    rowz2 = jnp.where(ilo8, zero, m1[1])

    def sub(ch, p):
        pltpu.make_async_copy(x_hbm.at[idxw_v.at[p]],
                              rows2_v.at[p], semg[p]).wait()
        nxt = 1 - p

        @pl.when(ch >= 1)
        def _():
            pltpu.make_async_copy(rows2_v.at[nxt],
                                  out_hbm.at[idxw_v.at[nxt]],
                                  semsc[nxt]).wait()

        @pl.when(ch + 1 < trip)
        def _():
            issue(ch + 1, nxt)
        off = ch * CH

        def fix(i, _2):
            pv = lst_v[pl.ds(off + i, L)][0]
            g0 = (pv & 2) > 0
            g1 = (pv & 1) > 0
            v0 = rows2_v[p, i, pl.ds(0, L)]
            p1 = jnp.where(ilo8, v0, m0[0])
            p2 = jnp.where(ilo8, m1[0], jnp.where(g0, m0[0], v0))
            rows2_v[p, i, pl.ds(0, L)] = jnp.where(g1, p2, p1)
            rows2_v[p, i, pl.ds(L, L)] = jnp.where(g1, rowz2, rowz1)
            for k in range(2, NK):
                rows2_v[p, i, pl.ds(k * L, L)] = jnp.where(g1, m1[k], m0[k])
            return 0
        with jax.named_scope("fix"):
            lax.fori_loop(0, CH, fix, 0, unroll=2)
        pltpu.async_copy(rows2_v.at[p], out_hbm.at[idxw_v.at[p]], semsc[p])

    def pair(t, _):
        ch0 = 2 * t
        sub(ch0, 0)

        @pl.when(ch0 + 1 < trip)
        def _():
            sub(ch0 + 1, 1)
        return 0
    with jax.named_scope("chunks"):
        lax.fori_loop(0, (trip + 1) >> 1, pair, 0)

    @pl.when((trip & 1) == 1)
    def _():
        pltpu.make_async_copy(rows2_v.at[0], out_hbm.at[idxw_v.at[0]],
                              sems0).wait()

    @pl.when(((trip & 1) == 0) & (trip >= 2))
    def _():
        pltpu.make_async_copy(rows2_v.at[1], out_hbm.at[idxw_v.at[1]],
                              sems1).wait()


def _make_kernels():
    mesh = plsc.VectorSubcoreMesh(core_axis_name="c", subcore_axis_name="s")
    params = pltpu.CompilerParams(needs_layout_passes=False)
    ka = pl.kernel(
        _body_a,
        out_type=(
            jax.ShapeDtypeStruct((NC, PW), jnp.float32),
            jax.ShapeDtypeStruct((N,), jnp.float32),
            jax.ShapeDtypeStruct((N,), jnp.float32),
        ),
        mesh=mesh,
        compiler_params=params,
        scratch_types=[
            pltpu.VMEM((NCH, CH), jnp.int32),
            pltpu.VMEM((NCH, CH), jnp.int32),
            pltpu.VMEM((GCH, CH), jnp.int32),
            pltpu.VMEM((GCH, CH), jnp.int32),
            pltpu.VMEM((CH,), jnp.float32),
            pltpu.VMEM((GCH, CH, D), jnp.float32),
            pltpu.VMEM((NS * L + L,), jnp.float32),
            pltpu.VMEM((2, CH), jnp.int32),
            pltpu.VMEM((PW,), jnp.float32),
            pltpu.VMEM((NS, PW), jnp.float32),
            pltpu.VMEM((3 * L, D), jnp.float32),
            pltpu.VMEM((N // NS,), jnp.float32),
            pltpu.VMEM((L, D), jnp.float32),
            pltpu.VMEM_SHARED((N,), jnp.float32),
            pltpu.VMEM_SHARED((N,), jnp.float32),
            pltpu.VMEM_SHARED((NS * 48, D), jnp.float32),
            pltpu.VMEM_SHARED((NS, PW), jnp.float32),
            pltpu.SemaphoreType.DMA,
            pltpu.SemaphoreType.DMA,
            pltpu.SemaphoreType.DMA,
        ],
    )
    kb = pl.kernel(
        _body_b,
        out_type=(),
        mesh=mesh,
        compiler_params=params,
        scratch_types=[
            pltpu.VMEM((4, CH, D), jnp.float32),
            pltpu.VMEM((NC, PW), jnp.float32),
            pltpu.VMEM((ROWS_PER_W + L,), jnp.float32),
            pltpu.VMEM((ROWS_PER_W + L,), jnp.float32),
            pltpu.VMEM((LSZ,), jnp.int32),
            pltpu.VMEM((4, CH), jnp.int32),
            pltpu.SemaphoreType.DMA,
            pltpu.SemaphoreType.DMA,
            pltpu.SemaphoreType.DMA,
            pltpu.SemaphoreType.DMA,
            pltpu.SemaphoreType.DMA,
            pltpu.SemaphoreType.DMA,
            pltpu.SemaphoreType.DMA,
            pltpu.SemaphoreType.DMA,
            pltpu.SemaphoreType.DMA,
        ],
    )
    return ka, kb


_sc_sums, _sc_fixup = _make_kernels()


def _copy_body(x_ref, o_ref):
    o_ref[...] = x_ref[...]


_tc_copy = pl.pallas_call(
    _copy_body,
    out_shape=jax.ShapeDtypeStruct((N, D), jnp.float32),
    grid=(64,),
    in_specs=[pl.BlockSpec((N // 64, D), lambda i: (i, 0))],
    out_specs=pl.BlockSpec((N // 64, D), lambda i: (i, 0)),
)


@jax.jit
def _selective_filter(x, idx0, idx1):
    out_ref = jax.new_ref(_tc_copy(x))
    part, t0, t1 = _sc_sums(x, idx0, idx1)
    _sc_fixup(x, part, t0, t1, out_ref)
    return jax.freeze(out_ref)


def kernel(input, idx0, idx1):
    return _selective_filter(input, idx0, idx1)


# speedup vs baseline: 1.3116x; 1.3116x over previous
import jax
import jax.numpy as jnp
from jax import lax
from jax.experimental import pallas as pl
from jax.experimental.pallas import tpu as pltpu
from jax.experimental.pallas import tpu_sc as plsc

N = 65536
D = 128
B = 8192
NC = 2
NS = 16
L = 16
NW = NC * NS
ROWS_PER_W = N // NW
POS_PER_TILE = B // NS
CH = 128
NCH = POS_PER_TILE // CH
GPOS = B // NW
GCH = GPOS // CH
NK = D // L
LSZ = ROWS_PER_W + 2 * CH
PW = 512


def _body_a(x_hbm, idx0_hbm, idx1_hbm,
            part_hbm, t0_hbm, t1_hbm,
            idx0_v, idx1_v, g0i_v, g1i_v, ones_v, rows2_v, f0_v,
            sel_v, partial_v, pall_v, accv_v, zbuf_v, zbuf2_v,
            table0, table1, accspm, pspm, semg0, semg1, sem2):
    c = lax.axis_index("c")
    s = lax.axis_index("s")
    wid = s * NC + c

    def zb(i, _):
        zbuf_v[pl.ds(i * L, L)] = jnp.zeros((L,), jnp.float32)
        return 0
    lax.fori_loop(0, (N // NS) // L, zb, 0, unroll=4)
    pltpu.sync_copy(zbuf_v, table0.at[pl.ds(s * (N // NS), N // NS)])
    pltpu.sync_copy(zbuf_v, table1.at[pl.ds(s * (N // NS), N // NS)])
    def zb2(i, _):
        for k in range(NK):
            zbuf2_v[i, pl.ds(k * L, L)] = jnp.zeros((L,), jnp.float32)
        return 0
    lax.fori_loop(0, L, zb2, 0)
    for r in range(3):
        pltpu.sync_copy(zbuf2_v, accspm.at[pl.ds(48 * s + 16 * r, 16)])
    for k in range(CH // L):
        ones_v[pl.ds(k * L, L)] = jnp.ones((L,), jnp.float32)
    plsc.subcore_barrier()

    gbase = B // NC * c + GPOS * s
    idx_copies = []
    for j in range(NCH):
        idx_copies.append((idx0_hbm.at[pl.ds(POS_PER_TILE * s + CH * j, CH)],
                           idx0_v.at[j]))
        idx_copies.append((idx1_hbm.at[pl.ds(POS_PER_TILE * s + CH * j, CH)],
                           idx1_v.at[j]))
    for j in range(GCH):
        idx_copies.append((idx0_hbm.at[pl.ds(gbase + CH * j, CH)],
                           g0i_v.at[j]))
        idx_copies.append((idx1_hbm.at[pl.ds(gbase + CH * j, CH)],
                           g1i_v.at[j]))
    for src, dst in idx_copies:
        pltpu.async_copy(src, dst, semg0)
    for src, dst in idx_copies:
        pltpu.make_async_copy(src, dst, semg0).wait()

    add_copies = []
    for j in range(NCH):
        add_copies.append((ones_v, table0.at[idx0_v.at[j]]))
        add_copies.append((ones_v, table1.at[idx1_v.at[j]]))
    for src, dst in add_copies:
        pltpu.async_copy(src, dst, semg1, add=True)
    for src, dst in add_copies:
        pltpu.make_async_copy(src, dst, semg1).wait()
    plsc.subcore_barrier()

    pltpu.async_copy(table0.at[pl.ds(wid * ROWS_PER_W, ROWS_PER_W)],
                     t0_hbm.at[pl.ds(wid * ROWS_PER_W, ROWS_PER_W)], sem2)
    pltpu.async_copy(table1.at[pl.ds(wid * ROWS_PER_W, ROWS_PER_W)],
                     t1_hbm.at[pl.ds(wid * ROWS_PER_W, ROWS_PER_W)], sem2)
    sems = [semg0, semg1]

    lanei = lax.iota(jnp.int32, L)
    for k in range(CH // L):
        sel_v[0, pl.ds(k * L, L)] = jnp.full((L,), 48 * s, jnp.int32) + lanei
    for j in range(GCH):
        pltpu.async_copy(x_hbm.at[g0i_v.at[j]], rows2_v.at[j], sems[j])
    for j in range(GCH):
        pltpu.make_async_copy(x_hbm.at[g0i_v.at[j]],
                              rows2_v.at[j], sems[j]).wait()
        pltpu.sync_copy(rows2_v.at[j], accspm.at[sel_v.at[0]], add=True)

    for j in range(GCH):
        pltpu.async_copy(x_hbm.at[g1i_v.at[j]], rows2_v.at[j], sems[j])
    cvec = jnp.zeros((L,), jnp.float32)
    one = jnp.ones((L,), jnp.float32)
    zerov = jnp.zeros((L,), jnp.float32)
    hbase = jnp.full((L,), 48 * s + 16, jnp.int32) + lanei
    mbase = jnp.full((L,), 48 * s + 32, jnp.int32) + lanei
    for j in range(GCH):
        pltpu.sync_copy(table0.at[g1i_v.at[j]], f0_v.at[pl.ds(0, CH)])
        for k in range(CH // L):
            hit = f0_v[pl.ds(k * L, L)] > 0.0
            sel_v[1, pl.ds(k * L, L)] = jnp.where(hit, hbase, mbase)
            cvec = cvec + jnp.where(hit, one, zerov)
        pltpu.make_async_copy(x_hbm.at[g1i_v.at[j]],
                              rows2_v.at[j], sems[j]).wait()
        pltpu.sync_copy(rows2_v.at[j], accspm.at[sel_v.at[1]], add=True)

    pltpu.sync_copy(accspm.at[pl.ds(48 * s, 48)], accv_v)

    def redt(t, carry):
        res = []
        for g in range(3):
            for k in range(NK):
                res.append(carry[g * NK + k]
                           + accv_v[16 * g + t, pl.ds(k * L, L)])
        return tuple(res)
    tot = lax.fori_loop(0, L, redt,
                        tuple([jnp.zeros((L,), jnp.float32)] * (3 * NK)),
                        unroll=2)
    for k in range(NK):
        partial_v[pl.ds(k * L, L)] = tot[k]
        partial_v[pl.ds(D + k * L, L)] = tot[NK + k] + tot[2 * NK + k]
        partial_v[pl.ds(2 * D + k * L, L)] = tot[NK + k]
    partial_v[pl.ds(3 * D, L)] = cvec
    partial_v[pl.ds(3 * D + L, L)] = jnp.zeros((L,), jnp.float32)
    pltpu.sync_copy(partial_v, pspm.at[s])
    plsc.subcore_barrier()

    @pl.when(s == 0)
    def _():
        pltpu.sync_copy(pspm, pall_v)

        def redp(t, carry):
            return tuple(carry[k] + pall_v[t, pl.ds(k * L, L)]
                         for k in range(3 * NK + 1))
        tot2 = lax.fori_loop(0, NS, redp,
                             tuple([jnp.zeros((L,), jnp.float32)]
                                   * (3 * NK + 1)), unroll=2)
        for k in range(3 * NK + 1):
            partial_v[pl.ds(k * L, L)] = tot2[k]
        partial_v[pl.ds((3 * NK + 1) * L, L)] = jnp.zeros((L,), jnp.float32)
        pltpu.sync_copy(partial_v, part_hbm.at[c])

    pltpu.make_async_copy(
        table0.at[pl.ds(wid * ROWS_PER_W, ROWS_PER_W)],
        t0_hbm.at[pl.ds(wid * ROWS_PER_W, ROWS_PER_W)], sem2).wait()
    pltpu.make_async_copy(
        table1.at[pl.ds(wid * ROWS_PER_W, ROWS_PER_W)],
        t1_hbm.at[pl.ds(wid * ROWS_PER_W, ROWS_PER_W)], sem2).wait()


def _body_b(x_hbm, part_hbm, t0_hbm, t1_hbm, out_hbm,
            rows2_v, pall2_v, fl0_v, fl1_v, lst_v, idxw_v,
            semg0, semg1, semg2, semg3, sems0, sems1, sems2, sems3, semprt):
    c = lax.axis_index("c")
    s = lax.axis_index("s")
    wid = s * NC + c
    base = wid * ROWS_PER_W

    fl_copies = [
        (t0_hbm.at[pl.ds(base, ROWS_PER_W)], fl0_v.at[pl.ds(0, ROWS_PER_W)]),
        (t1_hbm.at[pl.ds(base, ROWS_PER_W)], fl1_v.at[pl.ds(0, ROWS_PER_W)]),
    ]
    pltpu.async_copy(part_hbm, pall2_v, semprt)
    for src, dst in fl_copies:
        pltpu.async_copy(src, dst, semg0)
    for src, dst in fl_copies:
        pltpu.make_async_copy(src, dst, semg0).wait()

    lane = lax.iota(jnp.int32, L)

    def bld(g, cnt):
        f0 = fl0_v[pl.ds(g * L, L)]
        f1 = fl1_v[pl.ds(g * L, L)]
        m = (f0 > 0.0) | (f1 > 0.0)
        key = jnp.where(m, 0, 1).astype(jnp.int32)
        ids = jnp.full((L,), base + g * L, jnp.int32) + lane
        val = ((ids << 2)
               | jnp.where(f0 > 0.0, 2, 0).astype(jnp.int32)
               | jnp.where(f1 > 0.0, 1, 0).astype(jnp.int32))
        _, vs = plsc.sort_key_val(key, val)
        lst_v[pl.ds(cnt, L)] = vs
        return cnt + plsc.all_reduce_population_count(m)[0]
    with jax.named_scope("bld"):
        cnt = lax.fori_loop(0, ROWS_PER_W // L, bld, jnp.int32(0), unroll=2)

    lastp = jnp.maximum(cnt - 1, 0)
    lid = jnp.full((L,), lst_v[pl.ds(lastp, L)][0], jnp.int32)
    for k in range(NK):
        lst_v[pl.ds(cnt + k * L, L)] = lid

    trip = lax.shift_right_logical(cnt + (CH - 1), 7)
    semg = [semg0, semg1, semg2, semg3]
    semsc = [sems0, sems1, sems2, sems3]

    def issue(ch, p):
        off = ch * CH
        for k in range(NK):
            idxw_v[p, pl.ds(k * L, L)] = lax.shift_right_logical(
                lst_v[pl.ds(off + k * L, L)], 2)
        pltpu.async_copy(x_hbm.at[idxw_v.at[p]], rows2_v.at[p], semg[p])

    @pl.when(trip > 0)
    def _():
        issue(0, 0)

    pltpu.make_async_copy(part_hbm, pall2_v, semprt).wait()

    def two(off):
        return pall2_v[0, pl.ds(off, L)] + pall2_v[1, pl.ds(off, L)]
    sum0 = [two(k * L) for k in range(NK)]
    s1x = [two(D + k * L) for k in range(NK)]
    sh = [two(2 * D + k * L) for k in range(NK)]
    cl = two(3 * D)
    cv = jnp.full((L,), lax.reduce_sum(cl, axes=(0,)), jnp.float32)

    inv_b = jnp.float32(1.0 / B)
    m0 = [sum0[k] * inv_b for k in range(NK)]
    ilo8 = lax.iota(jnp.int32, L) < 8
    zero = jnp.zeros((L,), jnp.float32)
    m1 = []
    for k in range(NK):
        if k == 0:
            r = jnp.where(ilo8, sh[0], cv * m0[0])
        elif k == 1:
            r = jnp.where(ilo8, zero, cv * m0[1])
        else:
            r = cv * m0[k]
        m1.append((s1x[k] - sh[k] + r) * inv_b)
    rowz1 = jnp.where(ilo8, zero, m0[1])
    rowz2 = jnp.where(ilo8, zero, m1[1])

    def sub(ch, p):
        pltpu.make_async_copy(x_hbm.at[idxw_v.at[p]],
                              rows2_v.at[p], semg[p]).wait()
        nxt = 1 - p

        @pl.when(ch >= 1)
        def _():
            pltpu.make_async_copy(rows2_v.at[nxt],
                                  out_hbm.at[idxw_v.at[nxt]],
                                  semsc[nxt]).wait()

        @pl.when(ch + 1 < trip)
        def _():
            issue(ch + 1, nxt)
        off = ch * CH

        def fix(i, _2):
            pv = lst_v[pl.ds(off + i, L)][0]
            g0 = (pv & 2) > 0
            g1 = (pv & 1) > 0
            v0 = rows2_v[p, i, pl.ds(0, L)]
            p1 = jnp.where(ilo8, v0, m0[0])
            p2 = jnp.where(ilo8, m1[0], jnp.where(g0, m0[0], v0))
            rows2_v[p, i, pl.ds(0, L)] = jnp.where(g1, p2, p1)
            rows2_v[p, i, pl.ds(L, L)] = jnp.where(g1, rowz2, rowz1)
            for k in range(2, NK):
                rows2_v[p, i, pl.ds(k * L, L)] = jnp.where(g1, m1[k], m0[k])
            return 0
        with jax.named_scope("fix"):
            lax.fori_loop(0, CH, fix, 0, unroll=2)
        pltpu.async_copy(rows2_v.at[p], out_hbm.at[idxw_v.at[p]], semsc[p])

    def pair(t, _):
        ch0 = 2 * t
        sub(ch0, 0)

        @pl.when(ch0 + 1 < trip)
        def _():
            sub(ch0 + 1, 1)
        return 0
    with jax.named_scope("chunks"):
        lax.fori_loop(0, (trip + 1) >> 1, pair, 0)

    @pl.when((trip & 1) == 1)
    def _():
        pltpu.make_async_copy(rows2_v.at[0], out_hbm.at[idxw_v.at[0]],
                              sems0).wait()

    @pl.when(((trip & 1) == 0) & (trip >= 2))
    def _():
        pltpu.make_async_copy(rows2_v.at[1], out_hbm.at[idxw_v.at[1]],
                              sems1).wait()


def _make_kernels():
    mesh = plsc.VectorSubcoreMesh(core_axis_name="c", subcore_axis_name="s")
    params = pltpu.CompilerParams(needs_layout_passes=False)
    ka = pl.kernel(
        _body_a,
        out_type=(
            jax.ShapeDtypeStruct((NC, PW), jnp.float32),
            jax.ShapeDtypeStruct((N,), jnp.float32),
            jax.ShapeDtypeStruct((N,), jnp.float32),
        ),
        mesh=mesh,
        compiler_params=params,
        scratch_types=[
            pltpu.VMEM((NCH, CH), jnp.int32),
            pltpu.VMEM((NCH, CH), jnp.int32),
            pltpu.VMEM((GCH, CH), jnp.int32),
            pltpu.VMEM((GCH, CH), jnp.int32),
            pltpu.VMEM((CH,), jnp.float32),
            pltpu.VMEM((GCH, CH, D), jnp.float32),
            pltpu.VMEM((NS * L + L,), jnp.float32),
            pltpu.VMEM((2, CH), jnp.int32),
            pltpu.VMEM((PW,), jnp.float32),
            pltpu.VMEM((NS, PW), jnp.float32),
            pltpu.VMEM((3 * L, D), jnp.float32),
            pltpu.VMEM((N // NS,), jnp.float32),
            pltpu.VMEM((L, D), jnp.float32),
            pltpu.VMEM_SHARED((N,), jnp.float32),
            pltpu.VMEM_SHARED((N,), jnp.float32),
            pltpu.VMEM_SHARED((NS * 48, D), jnp.float32),
            pltpu.VMEM_SHARED((NS, PW), jnp.float32),
            pltpu.SemaphoreType.DMA,
            pltpu.SemaphoreType.DMA,
            pltpu.SemaphoreType.DMA,
        ],
    )
    kb = pl.kernel(
        _body_b,
        out_type=(),
        mesh=mesh,
        compiler_params=params,
        scratch_types=[
            pltpu.VMEM((4, CH, D), jnp.float32),
            pltpu.VMEM((NC, PW), jnp.float32),
            pltpu.VMEM((ROWS_PER_W + L,), jnp.float32),
            pltpu.VMEM((ROWS_PER_W + L,), jnp.float32),
            pltpu.VMEM((LSZ,), jnp.int32),
            pltpu.VMEM((4, CH), jnp.int32),
            pltpu.SemaphoreType.DMA,
            pltpu.SemaphoreType.DMA,
            pltpu.SemaphoreType.DMA,
            pltpu.SemaphoreType.DMA,
            pltpu.SemaphoreType.DMA,
            pltpu.SemaphoreType.DMA,
            pltpu.SemaphoreType.DMA,
            pltpu.SemaphoreType.DMA,
            pltpu.SemaphoreType.DMA,
        ],
    )
    return ka, kb


_sc_sums, _sc_fixup = _make_kernels()


@jax.jit
def _selective_filter(x, idx0, idx1):
    out_ref = jax.new_ref(x)
    part, t0, t1 = _sc_sums(x, idx0, idx1)
    _sc_fixup(x, part, t0, t1, out_ref)
    return jax.freeze(out_ref)


def kernel(input, idx0, idx1):
    return _selective_filter(input, idx0, idx1)
